# Initial kernel scaffold; baseline (speedup 1.0000x reference)
import jax
import jax.numpy as jnp
from jax.experimental import pallas as pl


def kernel(output, target, one_hot):
    raise NotImplementedError("write your pallas kernel here")


# speedup vs baseline: 2.3673x; 2.3673x over previous
import functools

import jax
import jax.numpy as jnp
from jax.experimental import pallas as pl
from jax.experimental.pallas import tpu as pltpu

_LABEL_SMOOTHING = 0.1
_CONFIDENCE = 1.0 - _LABEL_SMOOTHING
_CB = 2048


def _xlogy(x):
    safe = jnp.where(x > 0, x, 1.0)
    return jnp.where(x > 0, x * jnp.log(safe), 0.0)


def _loss_body(tgt_ref, h_ref, out_ref, res_ref, acc_ref, *, b, v, smooth):
    j = pl.program_id(0)
    nb = pl.num_programs(0)

    @pl.when(j == 0)
    def _():
        acc_ref[0] = 0.0
        acc_ref[1] = 0.0
        acc_ref[2] = 0.0

    blk = out_ref[...]
    h = h_ref[...]
    tgt = tgt_ref[...]

    cols = jax.lax.broadcasted_iota(jnp.int32, (1, _CB), 1) + j * _CB
    valid = cols < v

    colsum = jnp.sum(blk, axis=0, keepdims=True)
    acc_ref[0] += jnp.sum(jnp.where(valid, colsum * h, 0.0))
    acc_ref[2] += jnp.sum(jnp.where(valid, _xlogy(h), 0.0))
    hit = cols == tgt
    acc_ref[1] += jnp.sum(jnp.where(hit, blk, 0.0))

    @pl.when(j == nb - 1)
    def _():
        const = b * (_xlogy_const(_CONFIDENCE) - _xlogy_const(smooth))
        res_ref[0, 0] = (
            b * acc_ref[2] + const
            - acc_ref[0] - (_CONFIDENCE - smooth) * acc_ref[1]
        )


def _xlogy_const(x):
    import math
    return x * math.log(x) if x > 0 else 0.0


@jax.jit
def kernel(output, target, one_hot):
    b, v = output.shape
    smooth = _LABEL_SMOOTHING / (v - 2)
    nb = pl.cdiv(v, _CB)
    tgt2d = target.astype(jnp.int32).reshape(b, 1)

    res = pl.pallas_call(
        functools.partial(_loss_body, b=b, v=v, smooth=smooth),
        grid=(nb,),
        in_specs=[
            pl.BlockSpec((b, 1), lambda j: (0, 0)),
            pl.BlockSpec((1, _CB), lambda j: (0, j)),
            pl.BlockSpec((b, _CB), lambda j: (0, j)),
        ],
        out_specs=pl.BlockSpec(memory_space=pltpu.SMEM),
        out_shape=jax.ShapeDtypeStruct((1, 1), jnp.float32),
        scratch_shapes=[pltpu.SMEM((3,), jnp.float32)],
        compiler_params=pltpu.CompilerParams(
            dimension_semantics=("arbitrary",),
        ),
    )(tgt2d, one_hot, output)
    return res[0, 0]


# speedup vs baseline: 2.3773x; 1.0042x over previous
import functools

import jax
import jax.numpy as jnp
from jax import lax
from jax.experimental import pallas as pl
from jax.experimental.pallas import tpu as pltpu
from jax.experimental.pallas import tpu_sc as plsc

_LABEL_SMOOTHING = 0.1
_CONFIDENCE = 1.0 - _LABEL_SMOOTHING
_CB = 2048
_NC = 2


def _xlogy(x):
    safe = jnp.where(x > 0, x, 1.0)
    return jnp.where(x > 0, x * jnp.log(safe), 0.0)


def _xlogy_const(x):
    import math
    return x * math.log(x) if x > 0 else 0.0


def _dense_body(h_ref, out_ref, res_ref, acc_ref, *, b, v, smooth):
    j = pl.program_id(0)
    nb = pl.num_programs(0)

    @pl.when(j == 0)
    def _():
        acc_ref[0] = 0.0
        acc_ref[1] = 0.0

    blk = out_ref[...]
    h = h_ref[...]
    cols = jax.lax.broadcasted_iota(jnp.int32, (1, _CB), 1) + j * _CB
    valid = cols < v

    colsum = jnp.sum(blk, axis=0, keepdims=True)
    acc_ref[0] += jnp.sum(jnp.where(valid, colsum * h, 0.0))
    acc_ref[1] += jnp.sum(jnp.where(valid, _xlogy(h), 0.0))

    @pl.when(j == nb - 1)
    def _():
        const = b * (_xlogy_const(_CONFIDENCE) - _xlogy_const(smooth))
        res_ref[0, 0] = b * acc_ref[1] + const - acc_ref[0]


def _dense_partial(one_hot, output):
    b, v = output.shape
    smooth = _LABEL_SMOOTHING / (v - 2)
    nb = pl.cdiv(v, _CB)
    return pl.pallas_call(
        functools.partial(_dense_body, b=b, v=v, smooth=smooth),
        grid=(nb,),
        in_specs=[
            pl.BlockSpec((1, _CB), lambda j: (0, j)),
            pl.BlockSpec((b, _CB), lambda j: (0, j)),
        ],
        out_specs=pl.BlockSpec(memory_space=pltpu.SMEM),
        out_shape=jax.ShapeDtypeStruct((1, 1), jnp.float32),
        scratch_shapes=[pltpu.SMEM((2,), jnp.float32)],
        compiler_params=pltpu.CompilerParams(
            dimension_semantics=("arbitrary",),
        ),
    )(one_hot, output)


def _sc_gather(output, cols128):
    b, v = output.shape
    per_core = b // _NC

    mesh = plsc.ScalarSubcoreMesh(axis_name="c", num_cores=_NC)

    @functools.partial(
        pl.kernel,
        out_type=jax.ShapeDtypeStruct((8 * b, 128), jnp.float32),
        mesh=mesh,
        scratch_types=[
            pltpu.SMEM((per_core,), jnp.int32),
            pltpu.SemaphoreType.DMA,
            pltpu.SemaphoreType.DMA,
        ],
    )
    def gather_kernel(out_hbm, c128_hbm, g_hbm, tbuf, sem_t, sem_g):
        cid = lax.axis_index("c")
        base = cid * per_core
        pltpu.async_copy(
            c128_hbm.at[pl.ds(base, per_core)], tbuf, sem_t
        ).wait()

        @pl.loop(0, per_core)
        def _(i):
            c128 = pl.multiple_of(tbuf[i], 128)
            row0 = pl.multiple_of(base + (i // 8) * 8, 8)
            pltpu.async_copy(
                out_hbm.at[pl.ds(row0, 8), pl.ds(c128, 128)],
                g_hbm.at[pl.ds(pl.multiple_of((base + i) * 8, 8), 8), :],
                sem_g,
            )

        pltpu.make_async_copy(
            out_hbm.at[pl.ds(0, 8 * per_core), pl.ds(0, 128)],
            g_hbm.at[pl.ds(8 * base, 8 * per_core), :],
            sem_g,
        ).wait()

    return gather_kernel(output, cols128)


def _combine_body(p_ref, off_ref, g_ref, res_ref, *, smooth):
    off = off_ref[...]
    g = g_ref[...]
    lane = jax.lax.broadcasted_iota(jnp.int32, g.shape, 1)
    gsum = jnp.sum(jnp.where(lane == off, g, 0.0))
    res_ref[0, 0] = p_ref[0, 0] - (_CONFIDENCE - smooth) * gsum


@jax.jit
def kernel(output, target, one_hot):
    b, v = output.shape
    smooth = _LABEL_SMOOTHING / (v - 2)
    tgt = target.astype(jnp.int32)
    cols128 = tgt & ~127
    i = jnp.arange(8 * b, dtype=jnp.int32)
    off = jnp.where(
        i % 8 == (i // 8) % 8, jnp.repeat(tgt & 127, 8), -1
    ).reshape(8 * b, 1)

    g = _sc_gather(output, cols128)
    part = _dense_partial(one_hot, output)

    res = pl.pallas_call(
        functools.partial(_combine_body, smooth=smooth),
        in_specs=[
            pl.BlockSpec(memory_space=pltpu.SMEM),
            pl.BlockSpec((8 * b, 1), lambda: (0, 0)),
            pl.BlockSpec((8 * b, 128), lambda: (0, 0)),
        ],
        out_specs=pl.BlockSpec(memory_space=pltpu.SMEM),
        out_shape=jax.ShapeDtypeStruct((1, 1), jnp.float32),
    )(part, off, g)
    return res[0, 0]


# speedup vs baseline: 2.3808x; 1.0015x over previous
import functools

import jax
import jax.numpy as jnp
from jax import lax
from jax.experimental import pallas as pl
from jax.experimental.pallas import tpu as pltpu
from jax.experimental.pallas import tpu_sc as plsc

_LABEL_SMOOTHING = 0.1
_CONFIDENCE = 1.0 - _LABEL_SMOOTHING
_CB = 2048
_NC = 2


def _xlogy(x):
    safe = jnp.where(x > 0, x, 1.0)
    return jnp.where(x > 0, x * jnp.log(safe), 0.0)


def _xlogy_const(x):
    import math
    return x * math.log(x) if x > 0 else 0.0


def _dense_body(h_ref, out_ref, res_ref, *, b, v):
    j = pl.program_id(0)

    blk = out_ref[...]
    h = h_ref[...]
    cols = jax.lax.broadcasted_iota(jnp.int32, (1, _CB), 1) + j * _CB
    valid = cols < v

    colsum = jnp.sum(blk, axis=0, keepdims=True)
    val = b * jnp.sum(jnp.where(valid, _xlogy(h), 0.0)) - jnp.sum(
        jnp.where(valid, colsum * h, 0.0)
    )
    res_ref[...] = jnp.full((1, 1, 128), val, dtype=jnp.float32)


def _dense_partial(one_hot, output):
    b, v = output.shape
    nb = pl.cdiv(v, _CB)
    return pl.pallas_call(
        functools.partial(_dense_body, b=b, v=v),
        grid=(nb,),
        in_specs=[
            pl.BlockSpec((1, _CB), lambda j: (0, j)),
            pl.BlockSpec((b, _CB), lambda j: (0, j)),
        ],
        out_specs=pl.BlockSpec((1, 1, 128), lambda j: (j, 0, 0)),
        out_shape=jax.ShapeDtypeStruct((nb, 1, 128), jnp.float32),
        compiler_params=pltpu.CompilerParams(
            dimension_semantics=("parallel",),
        ),
    )(one_hot, output)


def _sc_gather(output, cols128):
    b, v = output.shape
    per_core = b // _NC

    mesh = plsc.ScalarSubcoreMesh(axis_name="c", num_cores=_NC)

    @functools.partial(
        pl.kernel,
        out_type=jax.ShapeDtypeStruct((8 * b, 128), jnp.float32),
        mesh=mesh,
        scratch_types=[
            pltpu.SMEM((per_core,), jnp.int32),
            pltpu.SemaphoreType.DMA,
            pltpu.SemaphoreType.DMA,
        ],
    )
    def gather_kernel(out_hbm, c128_hbm, g_hbm, tbuf, sem_t, sem_g):
        cid = lax.axis_index("c")
        base = cid * per_core
        pltpu.async_copy(
            c128_hbm.at[pl.ds(base, per_core)], tbuf, sem_t
        ).wait()

        @pl.loop(0, per_core)
        def _(i):
            c128 = pl.multiple_of(tbuf[i], 128)
            row0 = pl.multiple_of(base + (i // 8) * 8, 8)
            pltpu.async_copy(
                out_hbm.at[pl.ds(row0, 8), pl.ds(c128, 128)],
                g_hbm.at[pl.ds(pl.multiple_of((base + i) * 8, 8), 8), :],
                sem_g,
            )

        pltpu.make_async_copy(
            out_hbm.at[pl.ds(0, 8 * per_core), pl.ds(0, 128)],
            g_hbm.at[pl.ds(8 * base, 8 * per_core), :],
            sem_g,
        ).wait()

    return gather_kernel(output, cols128)


def _combine_body(p_ref, off_ref, g_ref, res_ref, *, b, smooth):
    off = off_ref[...]
    g = g_ref[...]
    lane = jax.lax.broadcasted_iota(jnp.int32, g.shape, 1)
    gsum = jnp.sum(jnp.where(lane == off, g, 0.0))
    const = b * (_xlogy_const(_CONFIDENCE) - _xlogy_const(smooth))
    ptot = jnp.sum(p_ref[...]) / 128.0
    res_ref[0, 0] = ptot + const - (_CONFIDENCE - smooth) * gsum


@jax.jit
def kernel(output, target, one_hot):
    b, v = output.shape
    smooth = _LABEL_SMOOTHING / (v - 2)
    tgt = target.astype(jnp.int32)
    cols128 = tgt & ~127
    i = jnp.arange(8 * b, dtype=jnp.int32)
    off = jnp.where(
        i % 8 == (i // 8) % 8, jnp.repeat(tgt & 127, 8), -1
    ).reshape(8 * b, 1)

    g = _sc_gather(output, cols128)
    parts = _dense_partial(one_hot, output)
    parts = parts.reshape(parts.shape[0], 128)
    nb = parts.shape[0]

    res = pl.pallas_call(
        functools.partial(_combine_body, b=b, smooth=smooth),
        in_specs=[
            pl.BlockSpec((nb, 128), lambda: (0, 0)),
            pl.BlockSpec((8 * b, 1), lambda: (0, 0)),
            pl.BlockSpec((8 * b, 128), lambda: (0, 0)),
        ],
        out_specs=pl.BlockSpec(memory_space=pltpu.SMEM),
        out_shape=jax.ShapeDtypeStruct((1, 1), jnp.float32),
    )(parts, off, g)
    return res[0, 0]
